# baseline (speedup 1.0000x reference)
import numpy as np
import jax
import jax.numpy as jnp
from jax.experimental import pallas as pl
from jax.experimental.pallas import tpu as pltpu

_N = 12
_D = 64
_F = _N * _D
_C = 4
_CN = _C * _N


def _ico_anorm():
    faces = [(0, 11, 5), (0, 5, 1), (0, 1, 7), (0, 7, 10), (0, 10, 11),
             (1, 5, 9), (5, 11, 4), (11, 10, 2), (10, 7, 6), (7, 1, 8),
             (3, 9, 4), (3, 4, 2), (3, 2, 6), (3, 6, 8), (3, 8, 9),
             (4, 9, 5), (2, 4, 11), (6, 2, 10), (8, 6, 7), (9, 8, 1)]
    es = set()
    for f in faces:
        for i in range(3):
            a, b = f[i], f[(i + 1) % 3]
            es.add(tuple(sorted((a, b))))
    adj = np.zeros((_N, _N), dtype=np.float32)
    for a, b in es:
        adj[a, b] = 1.0
        adj[b, a] = 1.0
    deg = np.maximum(adj.sum(axis=1), 1.0)
    return adj / deg[:, None]


_SLOT_TO_VERT = np.array([4, 6, 7, 5, 9, 10, 1, 2, 5, 5, 4, 4])
_CELL_FOR_NODE = np.array([10, 10, 1, 4, 6, 10, 10, 5, 7, 3, 11, 11])


def _constants():
    anorm = _ico_anorm()
    eye = np.eye(_D, dtype=np.float32)
    t768 = np.tile(eye, (_N, 1))
    tilec = np.tile(eye, (1, _N))
    aexpf = np.kron(anorm.T, np.ones((_D, _D), np.float32))
    r48 = np.zeros((_CN, _C), np.float32)
    for c in range(_C):
        r48[c * _N:(c + 1) * _N, c] = 1.0
    rc = r48.T.copy()
    sel_in = (np.arange(_N)[:, None] == _CELL_FOR_NODE[None, :]).astype(np.float32)
    sel_inf = np.tile(np.repeat(sel_in, _D, axis=1), (_C, 1))
    sel_out = (np.arange(_N)[:, None] == _SLOT_TO_VERT[None, :]).astype(np.float32)
    sel_outf = np.tile(np.repeat(sel_out, _D, axis=0), (1, _C))
    return t768, tilec, aexpf, r48, rc, sel_inf, sel_outf


_T768, _TILEC, _AEXPF, _R48, _RC, _SEL_INF, _SEL_OUTF = _constants()
_L = 4


def _fwd(x_ref, wi_ref, bi_ref, wl_ref, bl_ref, wo_ref, bo_ref,
         t768_ref, tilec_ref, aexpf_ref, r48_ref, rc_ref, selin_ref,
         selout_ref, o_ref,
         m1_s, b1_s, wk_s, bk_s, m2_s, b2_s):
    @pl.when(pl.program_id(0) == 0)
    def _build():
        f32 = jnp.float32
        wi_t = jnp.dot(wi_ref[...], tilec_ref[...],
                       preferred_element_type=f32)
        m1 = jnp.dot(r48_ref[...], wi_t, preferred_element_type=f32)
        m1_s[...] = (m1 * selin_ref[...]).astype(jnp.bfloat16)
        b1_s[...] = jnp.dot(bi_ref[...], tilec_ref[...],
                            preferred_element_type=f32)
        bk_s[...] = jnp.dot(bl_ref[...], tilec_ref[...],
                            preferred_element_type=f32)[:, None, :]
        for k in range(_L):
            wlt = jnp.dot(t768_ref[...], wl_ref[k],
                          preferred_element_type=f32)
            wkk = jnp.dot(wlt, tilec_ref[...],
                          preferred_element_type=f32)
            wk_s[k] = (wkk * aexpf_ref[...]).astype(jnp.bfloat16)
        wo_r = jnp.dot(wo_ref[...], rc_ref[...],
                       preferred_element_type=f32)
        m2 = jnp.dot(t768_ref[...], wo_r, preferred_element_type=f32)
        m2_s[...] = (m2 * selout_ref[...]).astype(jnp.bfloat16)
        b2_s[...] = jnp.dot(bo_ref[...], rc_ref[...],
                            preferred_element_type=f32)

    rows = x_ref.shape[0]
    chunk = min(2048, rows)
    for c0 in range(0, rows, chunk):
        sl = pl.ds(c0, chunk)
        hn = jnp.dot(x_ref[sl, :].astype(jnp.bfloat16), m1_s[...],
                     preferred_element_type=jnp.float32) + b1_s[...]
        for k in range(_L):
            z = jnp.dot(hn.astype(jnp.bfloat16), wk_s[k],
                        preferred_element_type=jnp.float32) + bk_s[k]
            hn = hn + jnp.maximum(z, 0.0)
        o_ref[sl, :] = jnp.dot(hn.astype(jnp.bfloat16), m2_s[...],
                               preferred_element_type=jnp.float32) + b2_s[...]


def kernel(x, t, Wi, bi, Wl, bl, Wo, bo):
    del t
    b = x.shape[0]
    xf = x.reshape(b, _CN)
    bblk = b

    def full(*shape):
        return pl.BlockSpec(shape, lambda i: (0,) * len(shape))

    out = pl.pallas_call(
        _fwd,
        grid=(b // bblk,),
        in_specs=[
            pl.BlockSpec((bblk, _CN), lambda i: (i, 0)),
            full(_C, _D),
            full(1, _D),
            full(_L, _D, _D),
            full(_L, _D),
            full(_D, _C),
            full(1, _C),
            full(_F, _D),
            full(_D, _F),
            full(_F, _F),
            full(_CN, _C),
            full(_C, _CN),
            full(_CN, _F),
            full(_F, _CN),
        ],
        out_specs=pl.BlockSpec((bblk, _CN), lambda i: (i, 0)),
        out_shape=jax.ShapeDtypeStruct((b, _CN), jnp.float32),
        scratch_shapes=[
            pltpu.VMEM((_CN, _F), jnp.bfloat16),
            pltpu.VMEM((1, _F), jnp.float32),
            pltpu.VMEM((_L, _F, _F), jnp.bfloat16),
            pltpu.VMEM((_L, 1, _F), jnp.float32),
            pltpu.VMEM((_F, _CN), jnp.bfloat16),
            pltpu.VMEM((1, _CN), jnp.float32),
        ],
    )(xf, Wi, bi[None, :], Wl, bl, Wo, bo[None, :],
      jnp.asarray(_T768), jnp.asarray(_TILEC), jnp.asarray(_AEXPF),
      jnp.asarray(_R48), jnp.asarray(_RC), jnp.asarray(_SEL_INF),
      jnp.asarray(_SEL_OUTF))
    return out.reshape(x.shape)


# speedup vs baseline: 1.0598x; 1.0598x over previous
import numpy as np
import jax
import jax.numpy as jnp
from jax.experimental import pallas as pl
from jax.experimental.pallas import tpu as pltpu

_N = 12
_D = 64
_F = _N * _D
_C = 4
_CN = _C * _N


def _ico_anorm():
    faces = [(0, 11, 5), (0, 5, 1), (0, 1, 7), (0, 7, 10), (0, 10, 11),
             (1, 5, 9), (5, 11, 4), (11, 10, 2), (10, 7, 6), (7, 1, 8),
             (3, 9, 4), (3, 4, 2), (3, 2, 6), (3, 6, 8), (3, 8, 9),
             (4, 9, 5), (2, 4, 11), (6, 2, 10), (8, 6, 7), (9, 8, 1)]
    es = set()
    for f in faces:
        for i in range(3):
            a, b = f[i], f[(i + 1) % 3]
            es.add(tuple(sorted((a, b))))
    adj = np.zeros((_N, _N), dtype=np.float32)
    for a, b in es:
        adj[a, b] = 1.0
        adj[b, a] = 1.0
    deg = np.maximum(adj.sum(axis=1), 1.0)
    return adj / deg[:, None]


_SLOT_TO_VERT = np.array([4, 6, 7, 5, 9, 10, 1, 2, 5, 5, 4, 4])
_CELL_FOR_NODE = np.array([10, 10, 1, 4, 6, 10, 10, 5, 7, 3, 11, 11])


def _constants():
    anorm = _ico_anorm()
    eye = np.eye(_D, dtype=np.float32)
    t768 = np.tile(eye, (_N, 1))
    tilec = np.tile(eye, (1, _N))
    aexpf = np.kron(anorm.T, np.ones((_D, _D), np.float32))
    r48 = np.zeros((_CN, _C), np.float32)
    for c in range(_C):
        r48[c * _N:(c + 1) * _N, c] = 1.0
    rc = r48.T.copy()
    sel_in = (np.arange(_N)[:, None] == _CELL_FOR_NODE[None, :]).astype(np.float32)
    sel_inf = np.tile(np.repeat(sel_in, _D, axis=1), (_C, 1))
    sel_out = (np.arange(_N)[:, None] == _SLOT_TO_VERT[None, :]).astype(np.float32)
    sel_outf = np.tile(np.repeat(sel_out, _D, axis=0), (1, _C))
    return t768, tilec, aexpf, r48, rc, sel_inf, sel_outf


_T768, _TILEC, _AEXPF, _R48, _RC, _SEL_INF, _SEL_OUTF = _constants()
_L = 4


def _fwd(x_ref, wi_ref, bi_ref, wl_ref, bl_ref, wo_ref, bo_ref,
         t768_ref, tilec_ref, aexpf_ref, r48_ref, rc_ref, selin_ref,
         selout_ref, o_ref,
         m1_s, b1_s, wk_s, bk_s, m2_s, b2_s):
    @pl.when(pl.program_id(0) == 0)
    def _build():
        f32 = jnp.float32
        wi_t = jnp.dot(wi_ref[...], tilec_ref[...],
                       preferred_element_type=f32)
        m1 = jnp.dot(r48_ref[...], wi_t, preferred_element_type=f32)
        m1_s[...] = (m1 * selin_ref[...]).astype(jnp.bfloat16)
        b1_s[...] = jnp.dot(bi_ref[...], tilec_ref[...],
                            preferred_element_type=f32)
        bk_s[...] = jnp.dot(bl_ref[...], tilec_ref[...],
                            preferred_element_type=f32)[:, None, :]
        for k in range(_L):
            wlt = jnp.dot(t768_ref[...], wl_ref[k],
                          preferred_element_type=f32)
            wkk = jnp.dot(wlt, tilec_ref[...],
                          preferred_element_type=f32)
            wk_s[k] = (wkk * aexpf_ref[...]).astype(jnp.bfloat16)
        wo_r = jnp.dot(wo_ref[...], rc_ref[...],
                       preferred_element_type=f32)
        m2 = jnp.dot(t768_ref[...], wo_r, preferred_element_type=f32)
        m2_s[...] = (m2 * selout_ref[...]).astype(jnp.bfloat16)
        b2_s[...] = jnp.dot(bo_ref[...], rc_ref[...],
                            preferred_element_type=f32)

    rows = x_ref.shape[0]
    chunk = min(2048, rows)
    for c0 in range(0, rows, chunk):
        sl = pl.ds(c0, chunk)
        hn = (jnp.dot(x_ref[sl, :].astype(jnp.bfloat16), m1_s[...],
                      preferred_element_type=jnp.float32)
              + b1_s[...]).astype(jnp.bfloat16)
        for k in range(_L):
            z = jnp.dot(hn, wk_s[k],
                        preferred_element_type=jnp.float32) + bk_s[k]
            hn = hn + jnp.maximum(z, 0.0).astype(jnp.bfloat16)
        o_ref[sl, :] = jnp.dot(hn, m2_s[...],
                               preferred_element_type=jnp.float32) + b2_s[...]


def kernel(x, t, Wi, bi, Wl, bl, Wo, bo):
    del t
    b = x.shape[0]
    xf = x.reshape(b, _CN)
    bblk = min(4096, b)

    def full(*shape):
        return pl.BlockSpec(shape, lambda i: (0,) * len(shape))

    out = pl.pallas_call(
        _fwd,
        grid=(b // bblk,),
        in_specs=[
            pl.BlockSpec((bblk, _CN), lambda i: (i, 0)),
            full(_C, _D),
            full(1, _D),
            full(_L, _D, _D),
            full(_L, _D),
            full(_D, _C),
            full(1, _C),
            full(_F, _D),
            full(_D, _F),
            full(_F, _F),
            full(_CN, _C),
            full(_C, _CN),
            full(_CN, _F),
            full(_F, _CN),
        ],
        out_specs=pl.BlockSpec((bblk, _CN), lambda i: (i, 0)),
        out_shape=jax.ShapeDtypeStruct((b, _CN), jnp.float32),
        scratch_shapes=[
            pltpu.VMEM((_CN, _F), jnp.bfloat16),
            pltpu.VMEM((1, _F), jnp.float32),
            pltpu.VMEM((_L, _F, _F), jnp.bfloat16),
            pltpu.VMEM((_L, 1, _F), jnp.float32),
            pltpu.VMEM((_F, _CN), jnp.bfloat16),
            pltpu.VMEM((1, _CN), jnp.float32),
        ],
    )(xf, Wi, bi[None, :], Wl, bl, Wo, bo[None, :],
      jnp.asarray(_T768), jnp.asarray(_TILEC), jnp.asarray(_AEXPF),
      jnp.asarray(_R48), jnp.asarray(_RC), jnp.asarray(_SEL_INF),
      jnp.asarray(_SEL_OUTF))
    return out.reshape(x.shape)


# speedup vs baseline: 1.0680x; 1.0077x over previous
import numpy as np
import jax
import jax.numpy as jnp
from jax.experimental import pallas as pl
from jax.experimental.pallas import tpu as pltpu

_N = 12
_D = 64
_F = _N * _D
_C = 4
_CN = _C * _N


def _ico_anorm():
    faces = [(0, 11, 5), (0, 5, 1), (0, 1, 7), (0, 7, 10), (0, 10, 11),
             (1, 5, 9), (5, 11, 4), (11, 10, 2), (10, 7, 6), (7, 1, 8),
             (3, 9, 4), (3, 4, 2), (3, 2, 6), (3, 6, 8), (3, 8, 9),
             (4, 9, 5), (2, 4, 11), (6, 2, 10), (8, 6, 7), (9, 8, 1)]
    es = set()
    for f in faces:
        for i in range(3):
            a, b = f[i], f[(i + 1) % 3]
            es.add(tuple(sorted((a, b))))
    adj = np.zeros((_N, _N), dtype=np.float32)
    for a, b in es:
        adj[a, b] = 1.0
        adj[b, a] = 1.0
    deg = np.maximum(adj.sum(axis=1), 1.0)
    return adj / deg[:, None]


_SLOT_TO_VERT = np.array([4, 6, 7, 5, 9, 10, 1, 2, 5, 5, 4, 4])
_CELL_FOR_NODE = np.array([10, 10, 1, 4, 6, 10, 10, 5, 7, 3, 11, 11])


def _constants():
    anorm = _ico_anorm()
    eye = np.eye(_D, dtype=np.float32)
    t768 = np.tile(eye, (_N, 1))
    tilec = np.tile(eye, (1, _N))
    aexpf = np.kron(anorm.T, np.ones((_D, _D), np.float32))
    r48 = np.zeros((_CN, _C), np.float32)
    for c in range(_C):
        r48[c * _N:(c + 1) * _N, c] = 1.0
    rc = r48.T.copy()
    sel_in = (np.arange(_N)[:, None] == _CELL_FOR_NODE[None, :]).astype(np.float32)
    sel_inf = np.tile(np.repeat(sel_in, _D, axis=1), (_C, 1))
    sel_out = (np.arange(_N)[:, None] == _SLOT_TO_VERT[None, :]).astype(np.float32)
    sel_outf = np.tile(np.repeat(sel_out, _D, axis=0), (1, _C))
    return t768, tilec, aexpf, r48, rc, sel_inf, sel_outf


_T768, _TILEC, _AEXPF, _R48, _RC, _SEL_INF, _SEL_OUTF = _constants()
_L = 4


def _fwd(x_ref, wi_ref, bi_ref, wl_ref, bl_ref, wo_ref, bo_ref,
         t768_ref, tilec_ref, aexpf_ref, r48_ref, rc_ref, selin_ref,
         selout_ref, o_ref,
         m1_s, b1_s, wk_s, bk_s, m2_s, b2_s):
    @pl.when(pl.program_id(0) == 0)
    def _build():
        f32 = jnp.float32
        wi_t = jnp.dot(wi_ref[...], tilec_ref[...],
                       preferred_element_type=f32)
        m1 = jnp.dot(r48_ref[...], wi_t, preferred_element_type=f32)
        m1_s[...] = (m1 * selin_ref[...]).astype(jnp.bfloat16)
        b1_s[...] = jnp.dot(bi_ref[...], tilec_ref[...],
                            preferred_element_type=f32)
        bk_s[...] = jnp.dot(bl_ref[...], tilec_ref[...],
                            preferred_element_type=f32)[:, None, :]
        for k in range(_L):
            wlt = jnp.dot(t768_ref[...], wl_ref[k],
                          preferred_element_type=f32)
            wkk = jnp.dot(wlt, tilec_ref[...],
                          preferred_element_type=f32)
            wk_s[k] = (wkk * aexpf_ref[...]).astype(jnp.bfloat16)
        wo_r = jnp.dot(wo_ref[...], rc_ref[...],
                       preferred_element_type=f32)
        m2 = jnp.dot(t768_ref[...], wo_r, preferred_element_type=f32)
        m2_s[...] = (m2 * selout_ref[...]).astype(jnp.bfloat16)
        b2_s[...] = jnp.dot(bo_ref[...], rc_ref[...],
                            preferred_element_type=f32)

    rows = x_ref.shape[0]
    chunk = min(1024, rows)
    for c0 in range(0, rows, chunk):
        sl = pl.ds(c0, chunk)
        hn = (jnp.dot(x_ref[sl, :].astype(jnp.bfloat16), m1_s[...],
                      preferred_element_type=jnp.float32)
              + b1_s[...]).astype(jnp.bfloat16)
        for k in range(_L):
            z = jnp.dot(hn, wk_s[k],
                        preferred_element_type=jnp.float32) + bk_s[k]
            hn = hn + jnp.maximum(z, 0.0).astype(jnp.bfloat16)
        o_ref[sl, :] = jnp.dot(hn, m2_s[...],
                               preferred_element_type=jnp.float32) + b2_s[...]


def kernel(x, t, Wi, bi, Wl, bl, Wo, bo):
    del t
    b = x.shape[0]
    xf = x.reshape(b, _CN)
    bblk = min(4096, b)

    def full(*shape):
        return pl.BlockSpec(shape, lambda i: (0,) * len(shape))

    out = pl.pallas_call(
        _fwd,
        grid=(b // bblk,),
        in_specs=[
            pl.BlockSpec((bblk, _CN), lambda i: (i, 0)),
            full(_C, _D),
            full(1, _D),
            full(_L, _D, _D),
            full(_L, _D),
            full(_D, _C),
            full(1, _C),
            full(_F, _D),
            full(_D, _F),
            full(_F, _F),
            full(_CN, _C),
            full(_C, _CN),
            full(_CN, _F),
            full(_F, _CN),
        ],
        out_specs=pl.BlockSpec((bblk, _CN), lambda i: (i, 0)),
        out_shape=jax.ShapeDtypeStruct((b, _CN), jnp.float32),
        scratch_shapes=[
            pltpu.VMEM((_CN, _F), jnp.bfloat16),
            pltpu.VMEM((1, _F), jnp.float32),
            pltpu.VMEM((_L, _F, _F), jnp.bfloat16),
            pltpu.VMEM((_L, 1, _F), jnp.float32),
            pltpu.VMEM((_F, _CN), jnp.bfloat16),
            pltpu.VMEM((1, _CN), jnp.float32),
        ],
    )(xf, Wi, bi[None, :], Wl, bl, Wo, bo[None, :],
      jnp.asarray(_T768), jnp.asarray(_TILEC), jnp.asarray(_AEXPF),
      jnp.asarray(_R48), jnp.asarray(_RC), jnp.asarray(_SEL_INF),
      jnp.asarray(_SEL_OUTF))
    return out.reshape(x.shape)
